# baseline (speedup 1.0000x reference)
import jax
import jax.numpy as jnp
from jax.experimental import pallas as pl


def _body(x_ref, row_ref, col_ref, o_ref):
    o_ref[...] = (
        x_ref[...]
        + row_ref[...][None, :, None, :]
        + col_ref[...][None, None, :, :]
    )


def kernel(x, row_emb, col_emb):
    b, h, w, c = x.shape
    return pl.pallas_call(
        _body,
        grid=(b,),
        in_specs=[
            pl.BlockSpec((1, h, w, c), lambda i: (i, 0, 0, 0)),
            pl.BlockSpec((h, c), lambda i: (0, 0)),
            pl.BlockSpec((w, c), lambda i: (0, 0)),
        ],
        out_specs=pl.BlockSpec((1, h, w, c), lambda i: (i, 0, 0, 0)),
        out_shape=jax.ShapeDtypeStruct(x.shape, x.dtype),
    )(x, row_emb, col_emb)
